# baseline (speedup 1.0000x reference)
import functools

import jax
import jax.numpy as jnp
from jax import lax
from jax.experimental import pallas as pl
from jax.experimental.pallas import tpu as pltpu
from jax.experimental.pallas import tpu_sc as plsc

N = 100000
K = 128
C = 128

NC = 2
NS = 16
NW = NC * NS
L = 16

RPW = 3128
RPW_LAST = N - 31 * RPW
CH = 256
NBUF = 3
FULL = RPW // CH
FULL_LAST = RPW_LAST // CH
TAIL = RPW - FULL * CH
TAIL_LAST = RPW_LAST - FULL_LAST * CH
IDXBUF = RPW + 16


def _table_body(em_ref, pr_ref, post_ref, ll_ref):
  em = em_ref[...]
  pr = pr_ref[...]
  num = em * pr
  den = jnp.sum(num, axis=1, keepdims=True)
  post = num / den
  post_ref[...] = post
  ll_ref[...] = jnp.sum(post * jnp.log(num), axis=1, keepdims=True)


_table = pl.pallas_call(
    _table_body,
    out_shape=(
        jax.ShapeDtypeStruct((K, C), jnp.float32),
        jax.ShapeDtypeStruct((K, 1), jnp.float32),
    ),
)


_sc_mesh = plsc.VectorSubcoreMesh(
    core_axis_name="c", subcore_axis_name="s", num_cores=NC, num_subcores=NS)


@functools.partial(
    pl.kernel,
    out_type=(
        jax.ShapeDtypeStruct((N, C), jnp.float32),
        jax.ShapeDtypeStruct((NW, L), jnp.float32),
    ),
    mesh=_sc_mesh,
    compiler_params=pltpu.CompilerParams(needs_layout_passes=False),
    scratch_types=[
        pltpu.VMEM((IDXBUF,), jnp.int32),
        pltpu.VMEM_SHARED((K, C), jnp.float32),
        pltpu.VMEM((NBUF * CH, C), jnp.float32),
        pltpu.VMEM((C,), jnp.float32),
        pltpu.VMEM((L,), jnp.float32),
        pltpu.SemaphoreType.DMA,
        pltpu.SemaphoreType.DMA,
        pltpu.SemaphoreType.DMA,
    ],
)
def _sc_gather(post_hbm, ll_hbm, labels_hbm, out_hbm, llp_hbm,
               idx_v, table_sh, stage_v, ll_v, acc_v, ssem, gsem, osem):
  wid = lax.axis_index("s") * NC + lax.axis_index("c")
  row_base = wid * RPW
  is_last = wid == NW - 1

  @pl.when(lax.axis_index("s") == 0)
  def _():
    pltpu.sync_copy(post_hbm, table_sh)

  c_ll = pltpu.async_copy(ll_hbm, ll_v, ssem)

  @pl.when(jnp.logical_not(is_last))
  def _():
    pltpu.async_copy(labels_hbm.at[pl.ds(row_base, RPW)],
                     idx_v.at[pl.ds(0, RPW)], ssem)
    idx_v[pl.ds(RPW, L)] = jnp.zeros((L,), jnp.int32)

  @pl.when(is_last)
  def _():
    pltpu.async_copy(labels_hbm.at[pl.ds(row_base, RPW_LAST)],
                     idx_v.at[pl.ds(0, RPW_LAST)], ssem)
    idx_v[pl.ds(RPW_LAST, L)] = jnp.zeros((L,), jnp.int32)

  c_ll.wait()
  plsc.subcore_barrier()

  @pl.when(jnp.logical_not(is_last))
  def _():
    pltpu.make_async_copy(labels_hbm.at[pl.ds(row_base, RPW)],
                          idx_v.at[pl.ds(0, RPW)], ssem).wait()

  @pl.when(is_last)
  def _():
    pltpu.make_async_copy(labels_hbm.at[pl.ds(row_base, RPW_LAST)],
                          idx_v.at[pl.ds(0, RPW_LAST)], ssem).wait()

  iota = lax.iota(jnp.int32, L)
  acc_v[...] = jnp.zeros((L,), jnp.float32)

  def acc_ll_range(base, nv):
    a = acc_v[...]
    for v in range(nv):
      a = a + plsc.load_gather(ll_v, [idx_v[pl.ds(base + v * L, L)]])
    acc_v[...] = a

  def gather_chunk_start(t, sbuf, nrows):
    for h in range(0, nrows, 128):
      w = min(128, nrows - h)
      pltpu.async_copy(table_sh.at[idx_v.at[pl.ds(t * CH + h, w)]],
                       stage_v.at[pl.ds(sbuf + h, w)], gsem)

  def gather_chunk_wait(t, sbuf, nrows):
    for h in range(0, nrows, 128):
      w = min(128, nrows - h)
      pltpu.make_async_copy(table_sh.at[idx_v.at[pl.ds(t * CH + h, w)]],
                            stage_v.at[pl.ds(sbuf + h, w)], gsem).wait()

  nfull = jnp.where(is_last, FULL_LAST, FULL)

  gather_chunk_start(0, 0, CH)

  def chunk_body(t, carry):
    sbuf = (t % NBUF) * CH
    nsbuf = ((t + 1) % NBUF) * CH
    gather_chunk_wait(t, sbuf, CH)

    @pl.when(t >= 2)
    def _():
      pltpu.make_async_copy(stage_v.at[pl.ds(((t - 2) % NBUF) * CH, CH)],
                            out_hbm.at[pl.ds(row_base + (t - 2) * CH, CH)],
                            osem).wait()

    @pl.when(t + 1 < nfull)
    def _():
      gather_chunk_start(t + 1, nsbuf, CH)

    pltpu.async_copy(stage_v.at[pl.ds(sbuf, CH)],
                     out_hbm.at[pl.ds(row_base + t * CH, CH)], osem)
    acc_ll_range(t * CH, CH // L)
    return carry

  lax.fori_loop(0, nfull, chunk_body, 0)

  tbuf = (nfull % NBUF) * CH

  @pl.when(jnp.logical_not(is_last))
  def _():
    roff = FULL * CH
    pltpu.async_copy(table_sh.at[idx_v.at[pl.ds(roff, TAIL)]],
                     stage_v.at[pl.ds(tbuf, TAIL)], gsem).wait()
    pltpu.async_copy(stage_v.at[pl.ds(tbuf, TAIL)],
                     out_hbm.at[pl.ds(row_base + roff, TAIL)], osem)

  @pl.when(is_last)
  def _():
    roff = FULL_LAST * CH
    pltpu.async_copy(table_sh.at[idx_v.at[pl.ds(roff, 128)]],
                     stage_v.at[pl.ds(tbuf, 128)], gsem)
    pltpu.async_copy(table_sh.at[idx_v.at[pl.ds(roff + 128, TAIL_LAST - 128)]],
                     stage_v.at[pl.ds(tbuf + 128, TAIL_LAST - 128)], gsem)
    pltpu.make_async_copy(table_sh.at[idx_v.at[pl.ds(roff, 128)]],
                          stage_v.at[pl.ds(tbuf, 128)], gsem).wait()
    pltpu.make_async_copy(
        table_sh.at[idx_v.at[pl.ds(roff + 128, TAIL_LAST - 128)]],
        stage_v.at[pl.ds(tbuf + 128, TAIL_LAST - 128)], gsem).wait()
    pltpu.async_copy(stage_v.at[pl.ds(tbuf, TAIL_LAST)],
                     out_hbm.at[pl.ds(row_base + roff, TAIL_LAST)], osem)

  @pl.when(jnp.logical_not(is_last))
  def _():
    acc_ll_range(FULL * CH, TAIL // L)

  @pl.when(is_last)
  def _():
    acc_ll_range(FULL_LAST * CH, TAIL_LAST // L)

  rem_base = jnp.where(is_last, RPW_LAST - 8, RPW - 8)
  g = plsc.load_gather(ll_v, [idx_v[pl.ds(rem_base - 8, L)]])
  a = acc_v[...] + jnp.where(iota >= 8, g, jnp.zeros((L,), jnp.float32))
  acc_v[...] = a
  pltpu.sync_copy(acc_v, llp_hbm.at[wid])

  for d in (2, 1):
    td = nfull - d
    pltpu.make_async_copy(stage_v.at[pl.ds((td % NBUF) * CH, CH)],
                          out_hbm.at[pl.ds(row_base + td * CH, CH)],
                          osem).wait()

  @pl.when(jnp.logical_not(is_last))
  def _():
    pltpu.make_async_copy(stage_v.at[pl.ds(tbuf, TAIL)],
                          out_hbm.at[pl.ds(row_base + FULL * CH, TAIL)],
                          osem).wait()

  @pl.when(is_last)
  def _():
    pltpu.make_async_copy(stage_v.at[pl.ds(tbuf, TAIL_LAST)],
                          out_hbm.at[pl.ds(row_base + FULL_LAST * CH,
                                           TAIL_LAST)], osem).wait()


def kernel(labels, prior, emission):
  post, ll2d = _table(emission, prior.reshape(1, C))
  ll = ll2d.reshape(C)
  out, llp = _sc_gather(post, ll, labels.astype(jnp.int32))
  return jnp.sum(llp), out
